# baseline (speedup 1.0000x reference)
import functools
import math
import numpy as np
import jax
import jax.numpy as jnp
from jax import lax
from jax.experimental import pallas as pl
from jax.experimental.pallas import tpu as pltpu
from jax.experimental.pallas import tpu_sc as plsc

_C, _K, _D_IN, _M, _N = 64, 16, 1024, 1024, 8192
_NODES = _K - 1
_DEPTH = 4
_CK = _C * _K
_EPS = 1e-8

_node_map = np.zeros((_DEPTH, _K), dtype=np.int32)
_sign_map = np.zeros((_DEPTH, _K), dtype=np.float32)
for _k in range(_K):
    _node = 0
    for _l in range(_DEPTH):
        _bit = (_k >> (_DEPTH - 1 - _l)) & 1
        _node_map[_l, _k] = _node
        _sign_map[_l, _k] = 1.0 if _bit else -1.0
        _node = 2 * _node + 1 + _bit

_bp = np.zeros((_K, _NODES), dtype=np.float32)
_bm = np.zeros((_K, _NODES), dtype=np.float32)
for _k in range(_K):
    for _l in range(_DEPTH):
        _j = _node_map[_l, _k]
        if _sign_map[_l, _k] > 0:
            _bp[_k, _j] = 1.0
        else:
            _bm[_k, _j] = 1.0
_wp = np.zeros((_C, _NODES, _C, _K), dtype=np.float32)
_wm = np.zeros((_C, _NODES, _C, _K), dtype=np.float32)
for _c in range(_C):
    _wp[_c, :, _c, :] = _bp.T
    _wm[_c, :, _c, :] = _bm.T
_wp = _wp.reshape(_C * _NODES, _CK)
_wm = _wm.reshape(_C * _NODES, _CK)

_NUM_WORKERS = 32
_N_SC = 2048
_RBLK = 16
_UNROLL = 8
_VECS = _CK // 16


def _sc_encode_body(rows_per_w, i_hbm, dim_hbm, t_hbm, a_hbm, e_hbm,
                    dim_v, t_v, a_v, in_v, out_v):
    nc = 2
    wid = lax.axis_index("s") * nc + lax.axis_index("c")
    base = wid * rows_per_w

    pltpu.sync_copy(dim_hbm, dim_v)
    pltpu.sync_copy(t_hbm, t_v)
    pltpu.sync_copy(a_hbm, a_v)

    def block_body(b, carry):
        r0 = base + b * _RBLK
        pltpu.sync_copy(i_hbm.at[pl.ds(r0, _RBLK)], in_v)

        def v_body(v, carry2):
            col = v * 16
            idx = [dim_v[d, pl.ds(col, 16)] for d in range(_DEPTH)]
            tt = [t_v[d, pl.ds(col, 16)] for d in range(_DEPTH)]
            aa = [a_v[d, pl.ds(col, 16)] for d in range(_DEPTH)]

            def r_body(ru, carry3):
                r = ru * _UNROLL
                xs = []
                for u in range(_UNROLL):
                    roff = jnp.full((16,), r + u, jnp.int32)
                    xs.append([plsc.load_gather(in_v, [roff, idx[d]])
                               for d in range(_DEPTH)])
                ws = [[(tt[d] - xs[u][d]) * aa[d] for d in range(_DEPTH)]
                      for u in range(_UNROLL)]
                es = [[1.0 + jnp.exp(ws[u][d]) for d in range(_DEPTH)]
                      for u in range(_UNROLL)]
                dens = [(es[u][0] * es[u][1]) * (es[u][2] * es[u][3])
                        for u in range(_UNROLL)]
                recips = [1.0 / dens[u] for u in range(_UNROLL)]
                for u in range(_UNROLL):
                    out_v[r + u, pl.ds(col, 16)] = recips[u]
                return carry3

            return lax.fori_loop(0, _RBLK // _UNROLL, r_body, carry2)

        lax.fori_loop(0, _VECS, v_body, 0)
        pltpu.sync_copy(out_v, e_hbm.at[pl.ds(r0, _RBLK)])
        return carry

    lax.fori_loop(0, rows_per_w // _RBLK, block_body, 0)


def _sc_encode(I, dim_e, t_e, a_e, nrows):
    rows_per_w = nrows // _NUM_WORKERS
    mesh = plsc.VectorSubcoreMesh(core_axis_name="c", subcore_axis_name="s")
    kfn = functools.partial(
        pl.kernel,
        mesh=mesh,
        compiler_params=pltpu.CompilerParams(needs_layout_passes=False,
                                             use_tc_tiling_on_sc=True),
        out_type=jax.ShapeDtypeStruct((nrows, _CK), jnp.float32),
        scratch_types=[
            pltpu.VMEM((_DEPTH, _CK), jnp.int32),
            pltpu.VMEM((_DEPTH, _CK), jnp.float32),
            pltpu.VMEM((_DEPTH, _CK), jnp.float32),
            pltpu.VMEM((_RBLK, _D_IN), jnp.float32),
            pltpu.VMEM((_RBLK, _CK), jnp.float32),
        ],
    )(functools.partial(_sc_encode_body, rows_per_w))
    return kfn(I, dim_e, t_e, a_e)


_TC_TILE = 512
_SC_TILES = _N_SC // _TC_TILE
_CONTRACT_RHS1 = (((1,), (1,)), ((), ()))


def _split_dot(x, w_bf16, dims):
    hi = x.astype(jnp.bfloat16)
    lo = (x - hi.astype(jnp.float32)).astype(jnp.bfloat16)
    return (lax.dot_general(hi, w_bf16, dims,
                            preferred_element_type=jnp.float32)
            + lax.dot_general(lo, w_bf16, dims,
                              preferred_element_type=jnp.float32))


def _tc_fused_body(invt_ref, i_ref, sf_ref, tf_ref, wp_ref, wm_ref, lr_ref,
                   o_ref):
    invt = invt_ref[0, 0]
    h = _split_dot(i_ref[...], sf_ref[...], _CONTRACT_RHS1)
    z = (h - tf_ref[0, :][None, :]) * invt
    sig = jax.nn.sigmoid(z)
    logp = jnp.log(sig + _EPS)
    logm = jnp.log(1.0 - sig + _EPS)
    log_e = (
        jnp.dot(logp, wp_ref[...], preferred_element_type=jnp.float32)
        + jnp.dot(logm, wm_ref[...], preferred_element_type=jnp.float32)
    )
    e = jnp.exp(log_e).astype(jnp.bfloat16)
    o_ref[...] = lax.dot_general(e, lr_ref[...], _CONTRACT_RHS1,
                                 preferred_element_type=jnp.float32)


def _tc_fused(inv_t, I, s_flat, t_flat, lr_bf16):
    grid = ((_N - _N_SC) // _TC_TILE,)
    return pl.pallas_call(
        _tc_fused_body,
        grid=grid,
        in_specs=[
            pl.BlockSpec((1, 1), lambda i: (0, 0)),
            pl.BlockSpec((_TC_TILE, _D_IN), lambda i: (i + _SC_TILES, 0)),
            pl.BlockSpec((_C * _NODES, _D_IN), lambda i: (0, 0)),
            pl.BlockSpec((1, _C * _NODES), lambda i: (0, 0)),
            pl.BlockSpec((_C * _NODES, _CK), lambda i: (0, 0)),
            pl.BlockSpec((_C * _NODES, _CK), lambda i: (0, 0)),
            pl.BlockSpec((_M, _CK), lambda i: (0, 0)),
        ],
        out_specs=pl.BlockSpec((_TC_TILE, _M), lambda i: (i + _SC_TILES, 0)),
        out_shape=jax.ShapeDtypeStruct((_N, _M), jnp.float32),
    )(inv_t, I, s_flat, t_flat,
      jnp.asarray(_wp), jnp.asarray(_wm), lr_bf16)


def _tc_decode_body(e1_ref, lr_ref, prev_ref, o_ref):
    o_ref[...] = lax.dot_general(
        e1_ref[...].astype(jnp.bfloat16), lr_ref[...],
        _CONTRACT_RHS1, preferred_element_type=jnp.float32)


def _tc_decode(E1, lr_bf16, prev_out):
    grid = (_SC_TILES,)
    return pl.pallas_call(
        _tc_decode_body,
        grid=grid,
        in_specs=[
            pl.BlockSpec((_TC_TILE, _CK), lambda i: (i, 0)),
            pl.BlockSpec((_M, _CK), lambda i: (0, 0)),
            pl.BlockSpec(memory_space=pl.ANY),
        ],
        out_specs=pl.BlockSpec((_TC_TILE, _M), lambda i: (i, 0)),
        out_shape=jax.ShapeDtypeStruct((_N, _M), jnp.float32),
        input_output_aliases={2: 0},
    )(E1, lr_bf16, prev_out)


def kernel(I, T, L, S, B, dims, temp):
    node = jnp.asarray(_node_map)
    sign = jnp.asarray(_sign_map)
    inv_t = 1.0 / temp[0]

    dim_e = jnp.transpose(dims[:, node], (1, 0, 2)).reshape(_DEPTH, _CK)
    t_e = jnp.transpose(T[:, node], (1, 0, 2)).reshape(_DEPTH, _CK)
    a_e = jnp.broadcast_to((sign * inv_t)[:, None, :],
                           (_DEPTH, _C, _K)).reshape(_DEPTH, _CK)

    lr = L.reshape(_M, _CK).astype(jnp.bfloat16)
    s_flat = S.reshape(_C * _NODES, _D_IN).astype(jnp.bfloat16)
    t_flat = T.reshape(1, _C * _NODES)

    E1 = _sc_encode(I, dim_e.astype(jnp.int32), t_e, a_e, _N_SC)
    out = _tc_fused(inv_t.reshape(1, 1), I, s_flat, t_flat, lr)
    return _tc_decode(E1, lr, out)


# speedup vs baseline: 1.0730x; 1.0730x over previous
import functools
import math
import numpy as np
import jax
import jax.numpy as jnp
from jax import lax
from jax.experimental import pallas as pl
from jax.experimental.pallas import tpu as pltpu
from jax.experimental.pallas import tpu_sc as plsc

_C, _K, _D_IN, _M, _N = 64, 16, 1024, 1024, 8192
_NODES = _K - 1
_DEPTH = 4
_CK = _C * _K
_EPS = 1e-8

_node_map = np.zeros((_DEPTH, _K), dtype=np.int32)
_sign_map = np.zeros((_DEPTH, _K), dtype=np.float32)
for _k in range(_K):
    _node = 0
    for _l in range(_DEPTH):
        _bit = (_k >> (_DEPTH - 1 - _l)) & 1
        _node_map[_l, _k] = _node
        _sign_map[_l, _k] = 1.0 if _bit else -1.0
        _node = 2 * _node + 1 + _bit

_bp = np.zeros((_K, _NODES), dtype=np.float32)
_bm = np.zeros((_K, _NODES), dtype=np.float32)
for _k in range(_K):
    for _l in range(_DEPTH):
        _j = _node_map[_l, _k]
        if _sign_map[_l, _k] > 0:
            _bp[_k, _j] = 1.0
        else:
            _bm[_k, _j] = 1.0
_wp = np.zeros((_C, _NODES, _C, _K), dtype=np.float32)
_wm = np.zeros((_C, _NODES, _C, _K), dtype=np.float32)
for _c in range(_C):
    _wp[_c, :, _c, :] = _bp.T
    _wm[_c, :, _c, :] = _bm.T
_wp = _wp.reshape(_C * _NODES, _CK)
_wm = _wm.reshape(_C * _NODES, _CK)

_NUM_WORKERS = 32
_N_SC = 2048
_RBLK = 16
_UNROLL = 8
_VECS = _CK // 16


def _sc_encode_body(rows_per_w, i_hbm, dim_hbm, t_hbm, a_hbm, e_hbm,
                    dim_v, t_v, a_v, in_v, out_v):
    nc = 2
    wid = lax.axis_index("s") * nc + lax.axis_index("c")
    base = wid * rows_per_w

    pltpu.sync_copy(dim_hbm, dim_v)
    pltpu.sync_copy(t_hbm, t_v)
    pltpu.sync_copy(a_hbm, a_v)

    def block_body(b, carry):
        r0 = base + b * _RBLK
        pltpu.sync_copy(i_hbm.at[pl.ds(r0, _RBLK)], in_v)

        def v_body(v, carry2):
            col = v * 16
            idx = [dim_v[d, pl.ds(col, 16)] for d in range(_DEPTH)]
            tt = [t_v[d, pl.ds(col, 16)] for d in range(_DEPTH)]
            aa = [a_v[d, pl.ds(col, 16)] for d in range(_DEPTH)]

            def r_body(ru, carry3):
                r = ru * _UNROLL
                xs = []
                for u in range(_UNROLL):
                    roff = jnp.full((16,), r + u, jnp.int32)
                    xs.append([plsc.load_gather(in_v, [roff, idx[d]])
                               for d in range(_DEPTH)])
                ws = [[(tt[d] - xs[u][d]) * aa[d] for d in range(_DEPTH)]
                      for u in range(_UNROLL)]
                es = [[1.0 + jnp.exp(ws[u][d]) for d in range(_DEPTH)]
                      for u in range(_UNROLL)]
                dens = [(es[u][0] * es[u][1]) * (es[u][2] * es[u][3])
                        for u in range(_UNROLL)]
                recips = [1.0 / dens[u] for u in range(_UNROLL)]
                for u in range(_UNROLL):
                    out_v[r + u, pl.ds(col, 16)] = recips[u]
                return carry3

            return lax.fori_loop(0, _RBLK // _UNROLL, r_body, carry2)

        lax.fori_loop(0, _VECS, v_body, 0)
        pltpu.sync_copy(out_v, e_hbm.at[pl.ds(r0, _RBLK)])
        return carry

    lax.fori_loop(0, rows_per_w // _RBLK, block_body, 0)


def _sc_encode(I, dim_e, t_e, a_e, nrows):
    rows_per_w = nrows // _NUM_WORKERS
    mesh = plsc.VectorSubcoreMesh(core_axis_name="c", subcore_axis_name="s")
    kfn = functools.partial(
        pl.kernel,
        mesh=mesh,
        compiler_params=pltpu.CompilerParams(needs_layout_passes=False,
                                             use_tc_tiling_on_sc=True),
        out_type=jax.ShapeDtypeStruct((nrows, _CK), jnp.float32),
        scratch_types=[
            pltpu.VMEM((_DEPTH, _CK), jnp.int32),
            pltpu.VMEM((_DEPTH, _CK), jnp.float32),
            pltpu.VMEM((_DEPTH, _CK), jnp.float32),
            pltpu.VMEM((_RBLK, _D_IN), jnp.float32),
            pltpu.VMEM((_RBLK, _CK), jnp.float32),
        ],
    )(functools.partial(_sc_encode_body, rows_per_w))
    return kfn(I, dim_e, t_e, a_e)


_TC_TILE = 512
_SC_TILES = _N_SC // _TC_TILE
_CONTRACT_RHS1 = (((1,), (1,)), ((), ()))


def _split_dot(x, w_bf16, dims):
    hi = x.astype(jnp.bfloat16)
    lo = (x - hi.astype(jnp.float32)).astype(jnp.bfloat16)
    return (lax.dot_general(hi, w_bf16, dims,
                            preferred_element_type=jnp.float32)
            + lax.dot_general(lo, w_bf16, dims,
                              preferred_element_type=jnp.float32))


def _tc_fused_body(invt_ref, i_ref, sf_ref, tf_ref, wp_ref, wm_ref, lr_ref,
                   o_ref):
    invt = invt_ref[0, 0]
    h = lax.dot_general(i_ref[...], sf_ref[...], _CONTRACT_RHS1,
                        preferred_element_type=jnp.float32)
    z = (h - tf_ref[0, :][None, :]) * invt
    sig = jax.nn.sigmoid(z)
    logp = jnp.log(sig + _EPS)
    logm = jnp.log(1.0 - sig + _EPS)
    log_e = (
        jnp.dot(logp, wp_ref[...], preferred_element_type=jnp.float32)
        + jnp.dot(logm, wm_ref[...], preferred_element_type=jnp.float32)
    )
    e = jnp.exp(log_e).astype(jnp.bfloat16)
    o_ref[...] = lax.dot_general(e, lr_ref[...], _CONTRACT_RHS1,
                                 preferred_element_type=jnp.float32)


def _tc_fused(inv_t, I, s_flat, t_flat, lr_bf16):
    grid = ((_N - _N_SC) // _TC_TILE,)
    return pl.pallas_call(
        _tc_fused_body,
        grid=grid,
        in_specs=[
            pl.BlockSpec((1, 1), lambda i: (0, 0)),
            pl.BlockSpec((_TC_TILE, _D_IN), lambda i: (i + _SC_TILES, 0)),
            pl.BlockSpec((_C * _NODES, _D_IN), lambda i: (0, 0)),
            pl.BlockSpec((1, _C * _NODES), lambda i: (0, 0)),
            pl.BlockSpec((_C * _NODES, _CK), lambda i: (0, 0)),
            pl.BlockSpec((_C * _NODES, _CK), lambda i: (0, 0)),
            pl.BlockSpec((_M, _CK), lambda i: (0, 0)),
        ],
        out_specs=pl.BlockSpec((_TC_TILE, _M), lambda i: (i + _SC_TILES, 0)),
        out_shape=jax.ShapeDtypeStruct((_N, _M), jnp.float32),
    )(inv_t, I, s_flat, t_flat,
      jnp.asarray(_wp), jnp.asarray(_wm), lr_bf16)


def _tc_decode_body(e1_ref, lr_ref, prev_ref, o_ref):
    o_ref[...] = lax.dot_general(
        e1_ref[...].astype(jnp.bfloat16), lr_ref[...],
        _CONTRACT_RHS1, preferred_element_type=jnp.float32)


def _tc_decode(E1, lr_bf16, prev_out):
    grid = (_SC_TILES,)
    return pl.pallas_call(
        _tc_decode_body,
        grid=grid,
        in_specs=[
            pl.BlockSpec((_TC_TILE, _CK), lambda i: (i, 0)),
            pl.BlockSpec((_M, _CK), lambda i: (0, 0)),
            pl.BlockSpec(memory_space=pl.ANY),
        ],
        out_specs=pl.BlockSpec((_TC_TILE, _M), lambda i: (i, 0)),
        out_shape=jax.ShapeDtypeStruct((_N, _M), jnp.float32),
        input_output_aliases={2: 0},
    )(E1, lr_bf16, prev_out)


def kernel(I, T, L, S, B, dims, temp):
    node = jnp.asarray(_node_map)
    sign = jnp.asarray(_sign_map)
    inv_t = 1.0 / temp[0]

    dim_e = jnp.transpose(dims[:, node], (1, 0, 2)).reshape(_DEPTH, _CK)
    t_e = jnp.transpose(T[:, node], (1, 0, 2)).reshape(_DEPTH, _CK)
    a_e = jnp.broadcast_to((sign * inv_t)[:, None, :],
                           (_DEPTH, _C, _K)).reshape(_DEPTH, _CK)

    lr = L.reshape(_M, _CK).astype(jnp.bfloat16)
    s_flat = S.reshape(_C * _NODES, _D_IN)
    t_flat = T.reshape(1, _C * _NODES)

    E1 = _sc_encode(I, dim_e.astype(jnp.int32), t_e, a_e, _N_SC)
    out = _tc_fused(inv_t.reshape(1, 1), I, s_flat, t_flat, lr)
    return _tc_decode(E1, lr, out)


# speedup vs baseline: 1.0812x; 1.0076x over previous
import functools
import math
import numpy as np
import jax
import jax.numpy as jnp
from jax import lax
from jax.experimental import pallas as pl
from jax.experimental.pallas import tpu as pltpu
from jax.experimental.pallas import tpu_sc as plsc

_C, _K, _D_IN, _M, _N = 64, 16, 1024, 1024, 8192
_NODES = _K - 1
_DEPTH = 4
_CK = _C * _K
_EPS = 1e-8

_node_map = np.zeros((_DEPTH, _K), dtype=np.int32)
_sign_map = np.zeros((_DEPTH, _K), dtype=np.float32)
for _k in range(_K):
    _node = 0
    for _l in range(_DEPTH):
        _bit = (_k >> (_DEPTH - 1 - _l)) & 1
        _node_map[_l, _k] = _node
        _sign_map[_l, _k] = 1.0 if _bit else -1.0
        _node = 2 * _node + 1 + _bit

_bp = np.zeros((_K, _NODES), dtype=np.float32)
_bm = np.zeros((_K, _NODES), dtype=np.float32)
for _k in range(_K):
    for _l in range(_DEPTH):
        _j = _node_map[_l, _k]
        if _sign_map[_l, _k] > 0:
            _bp[_k, _j] = 1.0
        else:
            _bm[_k, _j] = 1.0
_wp = np.zeros((_C, _NODES, _C, _K), dtype=np.float32)
_wm = np.zeros((_C, _NODES, _C, _K), dtype=np.float32)
for _c in range(_C):
    _wp[_c, :, _c, :] = _bp.T
    _wm[_c, :, _c, :] = _bm.T
_wp = _wp.reshape(_C * _NODES, _CK)
_wm = _wm.reshape(_C * _NODES, _CK)

_NUM_WORKERS = 32
_N_SC = 2048
_RBLK = 16
_UNROLL = 8
_VECS = _CK // 16


def _sc_encode_body(rows_per_w, i_hbm, dim_hbm, t_hbm, a_hbm, e_hbm,
                    dim_v, t_v, a_v, in_v, out_v):
    nc = 2
    wid = lax.axis_index("s") * nc + lax.axis_index("c")
    base = wid * rows_per_w

    pltpu.sync_copy(dim_hbm, dim_v)
    pltpu.sync_copy(t_hbm, t_v)
    pltpu.sync_copy(a_hbm, a_v)

    def block_body(b, carry):
        r0 = base + b * _RBLK
        pltpu.sync_copy(i_hbm.at[pl.ds(r0, _RBLK)], in_v)

        def v_body(v, carry2):
            col = v * 16
            idx = [dim_v[d, pl.ds(col, 16)] for d in range(_DEPTH)]
            tt = [t_v[d, pl.ds(col, 16)] for d in range(_DEPTH)]
            aa = [a_v[d, pl.ds(col, 16)] for d in range(_DEPTH)]

            def r_body(ru, carry3):
                r = ru * _UNROLL
                xs = []
                for u in range(_UNROLL):
                    roff = jnp.full((16,), r + u, jnp.int32)
                    xs.append([plsc.load_gather(in_v, [roff, idx[d]])
                               for d in range(_DEPTH)])
                ws = [[(tt[d] - xs[u][d]) * aa[d] for d in range(_DEPTH)]
                      for u in range(_UNROLL)]
                es = [[1.0 + jnp.exp(ws[u][d]) for d in range(_DEPTH)]
                      for u in range(_UNROLL)]
                dens = [(es[u][0] * es[u][1]) * (es[u][2] * es[u][3])
                        for u in range(_UNROLL)]
                recips = [1.0 / dens[u] for u in range(_UNROLL)]
                for u in range(_UNROLL):
                    out_v[r + u, pl.ds(col, 16)] = recips[u]
                return carry3

            return lax.fori_loop(0, _RBLK // _UNROLL, r_body, carry2)

        lax.fori_loop(0, _VECS, v_body, 0)
        pltpu.sync_copy(out_v, e_hbm.at[pl.ds(r0, _RBLK)])
        return carry

    lax.fori_loop(0, rows_per_w // _RBLK, block_body, 0)


def _sc_encode(I, dim_e, t_e, a_e, nrows):
    rows_per_w = nrows // _NUM_WORKERS
    mesh = plsc.VectorSubcoreMesh(core_axis_name="c", subcore_axis_name="s")
    kfn = functools.partial(
        pl.kernel,
        mesh=mesh,
        compiler_params=pltpu.CompilerParams(needs_layout_passes=False,
                                             use_tc_tiling_on_sc=True),
        out_type=jax.ShapeDtypeStruct((nrows, _CK), jnp.float32),
        scratch_types=[
            pltpu.VMEM((_DEPTH, _CK), jnp.int32),
            pltpu.VMEM((_DEPTH, _CK), jnp.float32),
            pltpu.VMEM((_DEPTH, _CK), jnp.float32),
            pltpu.VMEM((_RBLK, _D_IN), jnp.float32),
            pltpu.VMEM((_RBLK, _CK), jnp.float32),
        ],
    )(functools.partial(_sc_encode_body, rows_per_w))
    return kfn(I, dim_e, t_e, a_e)


_TC_TILE = 1024
_SC_TILES = _N_SC // _TC_TILE
_CONTRACT_RHS1 = (((1,), (1,)), ((), ()))


def _split_dot(x, w_bf16, dims):
    hi = x.astype(jnp.bfloat16)
    lo = (x - hi.astype(jnp.float32)).astype(jnp.bfloat16)
    return (lax.dot_general(hi, w_bf16, dims,
                            preferred_element_type=jnp.float32)
            + lax.dot_general(lo, w_bf16, dims,
                              preferred_element_type=jnp.float32))


def _tc_fused_body(invt_ref, i_ref, sf_ref, tf_ref, wp_ref, wm_ref, lr_ref,
                   o_ref):
    invt = invt_ref[0, 0]
    h = lax.dot_general(i_ref[...], sf_ref[...], _CONTRACT_RHS1,
                        preferred_element_type=jnp.float32)
    z = (h - tf_ref[0, :][None, :]) * invt
    sig = jax.nn.sigmoid(z)
    logp = jnp.log(sig + _EPS)
    logm = jnp.log(1.0 - sig + _EPS)
    log_e = (
        jnp.dot(logp, wp_ref[...], preferred_element_type=jnp.float32)
        + jnp.dot(logm, wm_ref[...], preferred_element_type=jnp.float32)
    )
    e = jnp.exp(log_e).astype(jnp.bfloat16)
    o_ref[...] = lax.dot_general(e, lr_ref[...], _CONTRACT_RHS1,
                                 preferred_element_type=jnp.float32)


def _tc_fused(inv_t, I, s_flat, t_flat, lr_bf16):
    grid = ((_N - _N_SC) // _TC_TILE,)
    return pl.pallas_call(
        _tc_fused_body,
        grid=grid,
        in_specs=[
            pl.BlockSpec((1, 1), lambda i: (0, 0)),
            pl.BlockSpec((_TC_TILE, _D_IN), lambda i: (i + _SC_TILES, 0)),
            pl.BlockSpec((_C * _NODES, _D_IN), lambda i: (0, 0)),
            pl.BlockSpec((1, _C * _NODES), lambda i: (0, 0)),
            pl.BlockSpec((_C * _NODES, _CK), lambda i: (0, 0)),
            pl.BlockSpec((_C * _NODES, _CK), lambda i: (0, 0)),
            pl.BlockSpec((_M, _CK), lambda i: (0, 0)),
        ],
        out_specs=pl.BlockSpec((_TC_TILE, _M), lambda i: (i + _SC_TILES, 0)),
        out_shape=jax.ShapeDtypeStruct((_N, _M), jnp.float32),
    )(inv_t, I, s_flat, t_flat,
      jnp.asarray(_wp), jnp.asarray(_wm), lr_bf16)


def _tc_decode_body(e1_ref, lr_ref, prev_ref, o_ref):
    o_ref[...] = lax.dot_general(
        e1_ref[...].astype(jnp.bfloat16), lr_ref[...],
        _CONTRACT_RHS1, preferred_element_type=jnp.float32)


def _tc_decode(E1, lr_bf16, prev_out):
    grid = (_SC_TILES,)
    return pl.pallas_call(
        _tc_decode_body,
        grid=grid,
        in_specs=[
            pl.BlockSpec((_TC_TILE, _CK), lambda i: (i, 0)),
            pl.BlockSpec((_M, _CK), lambda i: (0, 0)),
            pl.BlockSpec(memory_space=pl.ANY),
        ],
        out_specs=pl.BlockSpec((_TC_TILE, _M), lambda i: (i, 0)),
        out_shape=jax.ShapeDtypeStruct((_N, _M), jnp.float32),
        input_output_aliases={2: 0},
    )(E1, lr_bf16, prev_out)


def kernel(I, T, L, S, B, dims, temp):
    node = jnp.asarray(_node_map)
    sign = jnp.asarray(_sign_map)
    inv_t = 1.0 / temp[0]

    dim_e = jnp.transpose(dims[:, node], (1, 0, 2)).reshape(_DEPTH, _CK)
    t_e = jnp.transpose(T[:, node], (1, 0, 2)).reshape(_DEPTH, _CK)
    a_e = jnp.broadcast_to((sign * inv_t)[:, None, :],
                           (_DEPTH, _C, _K)).reshape(_DEPTH, _CK)

    lr = L.reshape(_M, _CK).astype(jnp.bfloat16)
    s_flat = S.reshape(_C * _NODES, _D_IN)
    t_flat = T.reshape(1, _C * _NODES)

    E1 = _sc_encode(I, dim_e.astype(jnp.int32), t_e, a_e, _N_SC)
    out = _tc_fused(inv_t.reshape(1, 1), I, s_flat, t_flat, lr)
    return _tc_decode(E1, lr, out)
